# baseline (speedup 1.0000x reference)
import functools

import jax
import jax.numpy as jnp
from jax import lax
from jax.experimental import pallas as pl
from jax.experimental.pallas import tpu as pltpu
from jax.experimental.pallas import tpu_sc as plsc

NUM_TOKENS = 200
NBINS = 128
CAND = 512


def _imp_body(x_ref, imp_ref, thr_ref):
    xb = x_ref[0]
    s = jnp.sum(xb, axis=0)
    imp = jnp.dot(xb, s[:, None],
                  preferred_element_type=jnp.float32)
    impv = imp[:, 0]
    imp_ref[...] = impv[None, None, :]
    m1 = jnp.min(imp)
    m2 = jnp.max(imp)
    span0 = m2 - m1
    lo = m1 - 0.001 * span0 - 1e-6
    span = m2 - lo
    t = lax.broadcasted_iota(jnp.int32, (1, NBINS), 1).astype(jnp.float32)
    edges = lo + span * t * (1.0 / NBINS)
    impb = jnp.broadcast_to(imp, (imp.shape[0], NBINS))
    edb = jnp.broadcast_to(edges, (imp.shape[0], NBINS))
    cmp = jnp.where(impb > edb, 1.0, 0.0)
    e = jnp.sum(cmp, axis=0, keepdims=True)
    tstar = jnp.sum(jnp.where(e >= NUM_TOKENS, 1.0, 0.0)) - 1.0
    tc = lo + span * tstar * (1.0 / NBINS)
    thr_ref[...] = jnp.full((1, 1, 128), tc, jnp.float32)


def _importance(x):
    B, L, D = x.shape
    imp, thr = pl.pallas_call(
        _imp_body,
        grid=(B,),
        in_specs=[pl.BlockSpec((1, L, D), lambda b: (b, 0, 0))],
        out_specs=[
            pl.BlockSpec((1, 1, L), lambda b: (b, 0, 0)),
            pl.BlockSpec((1, 1, 128), lambda b: (b, 0, 0)),
        ],
        out_shape=[
            jax.ShapeDtypeStruct((B, 1, L), jnp.float32),
            jax.ShapeDtypeStruct((B, 1, 128), jnp.float32),
        ],
    )(x)
    return imp.reshape(B, L), thr.reshape(B, 128)


def _sc_select_gather(imp_flat, thr, x2d):
    BL, D = x2d.shape
    B = thr.shape[0]
    L = BL // B
    CHUNK = 16

    mesh = plsc.VectorSubcoreMesh(core_axis_name="c", subcore_axis_name="s")

    @functools.partial(
        pl.kernel,
        mesh=mesh,
        out_type=jax.ShapeDtypeStruct((4 * NUM_TOKENS, D), jnp.float32),
        scratch_types=[
            pltpu.VMEM((512,), jnp.float32),
            pltpu.VMEM((16,), jnp.float32),
            pltpu.VMEM((528,), jnp.float32),
            pltpu.VMEM((528,), jnp.int32),
            pltpu.VMEM((128,), jnp.int32),
            pltpu.VMEM((128,), jnp.int32),
            pltpu.VMEM((128,), jnp.int32),
            pltpu.VMEM((128,), jnp.int32),
            pltpu.VMEM((16,), jnp.int32),
            pltpu.VMEM((256,), jnp.int32),
            pltpu.VMEM((528,), jnp.float32),
            pltpu.VMEM((64,), jnp.int32),
            pltpu.VMEM((64,), jnp.int32),
            pltpu.VMEM((64,), jnp.int32),
            pltpu.VMEM((CHUNK,), jnp.int32),
            pltpu.VMEM((CHUNK, D), jnp.float32),
            pltpu.VMEM_SHARED((256,), jnp.int32),
            pltpu.VMEM_SHARED((2 * CAND + 16,), jnp.float32),
            pltpu.VMEM_SHARED((2 * CAND + 16,), jnp.int32),
            pltpu.VMEM_SHARED((416,), jnp.int32),
            pltpu.SemaphoreType.DMA,
        ],
    )
    def sc_kernel(imp_hbm, thr_hbm, x_hbm, out_hbm,
                  imp_v, thr_v, cval_v, cidx_v,
                  cpos0, cpos1, cpos2, cpos3,
                  cnt_v, counts_l, call_v, myidx_v, gi_v, fpos_v,
                  rowid_v, rows_v,
                  counts_sh, cval_sh, cidx_sh, final_sh, sem):
        c = lax.axis_index("c")
        s = lax.axis_index("s")
        b_local = s // 8
        part = s % 8
        b = c * 2 + b_local
        iota = lax.iota(jnp.int32, 16)
        one = jnp.int32(1)
        zero = jnp.int32(0)

        pltpu.sync_copy(imp_hbm.at[pl.ds(b * L + part * 512, 512)], imp_v)
        pltpu.sync_copy(thr_hbm.at[b, pl.ds(0, 16)], thr_v)
        thr_bc = thr_v[...]

        def cbody(j, off):
            u = imp_v[pl.ds(j * 16, 16)]
            mi = jnp.where(u > thr_bc, one, zero)
            base = part * 512 + j * 16
            rel = zero
            pv = jnp.zeros((16,), jnp.float32)
            pg = jnp.zeros((16,), jnp.int32)
            for t in range(16):
                sel = mi[t]
                relv = jnp.full((16,), rel, jnp.int32)
                selv = jnp.full((16,), sel, jnp.int32)
                hiti = jnp.where(iota == relv, selv, jnp.zeros((16,), jnp.int32))
                pv = jnp.where(hiti > 0, jnp.full((16,), u[t], jnp.float32), pv)
                pg = jnp.where(hiti > 0, jnp.full((16,), base + t, jnp.int32), pg)
                rel = rel + sel
            cval_v[pl.ds(off, 16)] = pv
            cidx_v[pl.ds(off, 16)] = pg
            return off + rel

        mycnt = lax.fori_loop(0, 32, cbody, zero)

        cnt_v[...] = jnp.full((16,), mycnt, jnp.int32)
        pltpu.sync_copy(cnt_v, counts_sh.at[pl.ds(s * 16, 16)])
        plsc.subcore_barrier()
        pltpu.sync_copy(counts_sh, counts_l)
        my_off = zero
        c_total = zero
        for t in range(8):
            g_t = counts_l[pl.ds((b_local * 8 + t) * 16, 16)][0]
            my_off = my_off + jnp.where(t < part, g_t, zero)
            c_total = c_total + g_t
        c_total = jnp.minimum(c_total, jnp.int32(CAND))

        cpos = (cpos0, cpos1, cpos2, cpos3)
        for k in range(4):
            for t2 in range(8):
                lpos = jnp.full((16,), k * 128 + t2 * 16, jnp.int32) + iota
                tgt = jnp.full((16,), b_local * CAND + my_off,
                               jnp.int32) + lpos
                zv16 = jnp.zeros((16,), jnp.int32)
                ov16 = jnp.full((16,), 1, jnp.int32)
                oki = jnp.where(lpos < jnp.full((16,), mycnt, jnp.int32),
                                ov16, zv16)
                oki = oki * jnp.where(
                    lpos < jnp.full((16,), CAND - my_off, jnp.int32),
                    ov16, zv16)
                cpos[k][pl.ds(t2 * 16, 16)] = jnp.where(
                    oki > 0, tgt, jnp.full((16,), 2 * CAND + s, jnp.int32))
        for k in range(4):
            pltpu.sync_copy(cval_v.at[pl.ds(k * 128, 128)],
                            cval_sh.at[cpos[k]])
            pltpu.sync_copy(cidx_v.at[pl.ds(k * 128, 128)],
                            cidx_sh.at[cpos[k]])
        plsc.subcore_barrier()

        pltpu.sync_copy(cval_sh.at[pl.ds(b_local * CAND, CAND)],
                        call_v.at[pl.ds(0, CAND)])
        pltpu.sync_copy(cidx_sh.at[pl.ds(b_local * CAND + part * 64, 64)],
                        myidx_v)
        myv = []
        mypos = []
        for i in range(4):
            pos = part * 64 + i * 16
            myv.append(call_v[pl.ds(pos, 16)])
            mypos.append(jnp.full((16,), pos, jnp.int32) + iota)

        def rbody(m, ranks):
            u = call_v[pl.ds(m, 16)]
            w = jnp.full((16,), u[0], jnp.float32)
            mvec = jnp.full((16,), m, jnp.int32)
            zv16 = jnp.zeros((16,), jnp.int32)
            ov16 = jnp.full((16,), 1, jnp.int32)
            res = []
            for i in range(4):
                lt = jnp.where(mvec < mypos[i], ov16, zv16)
                tie = jnp.where(w == myv[i], lt, zv16)
                add = jnp.where(w > myv[i], ov16, tie)
                res.append(ranks[i] + add)
            return tuple(res)

        zv = jnp.zeros((16,), jnp.int32)
        ranks = lax.fori_loop(0, c_total, rbody, (zv, zv, zv, zv))

        for i in range(4):
            rk = ranks[i]
            zv16 = jnp.zeros((16,), jnp.int32)
            ov16 = jnp.full((16,), 1, jnp.int32)
            vi = jnp.where(rk < jnp.full((16,), NUM_TOKENS, jnp.int32),
                           ov16, zv16)
            vi = vi * jnp.where(
                mypos[i] < jnp.full((16,), c_total, jnp.int32), ov16, zv16)
            fpos = jnp.full((16,), b_local * NUM_TOKENS, jnp.int32) + rk
            spos = jnp.where(vi > 0, fpos,
                             jnp.full((16,), 400 + s, jnp.int32))
            gi = (myidx_v[pl.ds(i * 16, 16)]
                  + jnp.full((16,), b * L, jnp.int32))
            fpos_v[pl.ds(i * 16, 16)] = spos
            gi_v[pl.ds(i * 16, 16)] = gi
        pltpu.sync_copy(gi_v, final_sh.at[fpos_v])
        plsc.subcore_barrier()

        for k in range(2):
            chunk = s + 16 * k

            @pl.when(chunk < 25)
            def _gather_chunk(chunk=chunk):
                pltpu.sync_copy(final_sh.at[pl.ds(chunk * CHUNK, CHUNK)],
                                rowid_v)
                pltpu.async_copy(x_hbm.at[rowid_v], rows_v, sem).wait()
                out_row = c * 2 * NUM_TOKENS + chunk * CHUNK
                pltpu.sync_copy(rows_v, out_hbm.at[pl.ds(out_row, CHUNK)])

    return sc_kernel(imp_flat, thr, x2d)


def _matmul_body(sel_ref, w_ref, b_ref, out_ref):
    a = sel_ref[...].astype(jnp.bfloat16)
    w = w_ref[...].astype(jnp.bfloat16)
    acc = lax.dot_general(
        a, w, (((1,), (1,)), ((), ())), preferred_element_type=jnp.float32)
    out_ref[...] = acc + b_ref[...]


def _project(sel2d, W, b):
    M, K = sel2d.shape
    N = W.shape[0]
    BN = 512
    return pl.pallas_call(
        _matmul_body,
        grid=(N // BN,),
        in_specs=[
            pl.BlockSpec((M, K), lambda j: (0, 0)),
            pl.BlockSpec((BN, K), lambda j: (j, 0)),
            pl.BlockSpec((1, BN), lambda j: (0, j)),
        ],
        out_specs=pl.BlockSpec((M, BN), lambda j: (0, j)),
        out_shape=jax.ShapeDtypeStruct((M, N), jnp.float32),
    )(sel2d, W, b[None, :])


def kernel(x, W, b):
    B, L, D = x.shape
    imp, thr = _importance(x)
    sel = _sc_select_gather(imp.reshape(B * L), thr, x.reshape(B * L, D))
    out = _project(sel, W, b)
    return out.reshape(B, NUM_TOKENS, W.shape[0])


# speedup vs baseline: 1.0634x; 1.0634x over previous
import functools

import jax
import jax.numpy as jnp
from jax import lax
from jax.experimental import pallas as pl
from jax.experimental.pallas import tpu as pltpu
from jax.experimental.pallas import tpu_sc as plsc

NUM_TOKENS = 200
NBINS = 128
CAND = 512


def _imp_body(x_ref, imp_ref, thr_ref):
    xb = x_ref[0]
    s = jnp.sum(xb, axis=0)
    imp = jnp.dot(xb, s[:, None],
                  preferred_element_type=jnp.float32)
    impv = imp[:, 0]
    imp_ref[...] = impv[None, None, :]
    m1 = jnp.min(imp)
    m2 = jnp.max(imp)
    span0 = m2 - m1
    lo = m1 - 0.001 * span0 - 1e-6
    span = m2 - lo
    t = lax.broadcasted_iota(jnp.int32, (1, NBINS), 1).astype(jnp.float32)
    edges = lo + span * t * (1.0 / NBINS)
    impb = jnp.broadcast_to(imp, (imp.shape[0], NBINS))
    edb = jnp.broadcast_to(edges, (imp.shape[0], NBINS))
    cmp = jnp.where(impb > edb, 1.0, 0.0)
    e = jnp.sum(cmp, axis=0, keepdims=True)
    tstar = jnp.sum(jnp.where(e >= NUM_TOKENS, 1.0, 0.0)) - 1.0
    tc = lo + span * tstar * (1.0 / NBINS)
    thr_ref[...] = jnp.full((1, 1, 128), tc, jnp.float32)


def _importance(x):
    B, L, D = x.shape
    imp, thr = pl.pallas_call(
        _imp_body,
        grid=(B,),
        in_specs=[pl.BlockSpec((1, L, D), lambda b: (b, 0, 0))],
        out_specs=[
            pl.BlockSpec((1, 1, L), lambda b: (b, 0, 0)),
            pl.BlockSpec((1, 1, 128), lambda b: (b, 0, 0)),
        ],
        out_shape=[
            jax.ShapeDtypeStruct((B, 1, L), jnp.float32),
            jax.ShapeDtypeStruct((B, 1, 128), jnp.float32),
        ],
    )(x)
    return imp.reshape(B, L), thr.reshape(B, 128)


def _sc_select_gather(imp_flat, thr, x2d):
    BL, D = x2d.shape
    B = thr.shape[0]
    L = BL // B
    CHUNK = 16

    mesh = plsc.VectorSubcoreMesh(core_axis_name="c", subcore_axis_name="s")

    @functools.partial(
        pl.kernel,
        mesh=mesh,
        out_type=jax.ShapeDtypeStruct((4 * NUM_TOKENS, D), jnp.float32),
        scratch_types=[
            pltpu.VMEM((512,), jnp.float32),
            pltpu.VMEM((16,), jnp.float32),
            pltpu.VMEM((528,), jnp.float32),
            pltpu.VMEM((528,), jnp.int32),
            pltpu.VMEM((128,), jnp.int32),
            pltpu.VMEM((128,), jnp.int32),
            pltpu.VMEM((128,), jnp.int32),
            pltpu.VMEM((128,), jnp.int32),
            pltpu.VMEM((16,), jnp.int32),
            pltpu.VMEM((256,), jnp.int32),
            pltpu.VMEM((528,), jnp.float32),
            pltpu.VMEM((64,), jnp.int32),
            pltpu.VMEM((64,), jnp.int32),
            pltpu.VMEM((64,), jnp.int32),
            pltpu.VMEM((CHUNK,), jnp.int32),
            pltpu.VMEM((CHUNK, D), jnp.float32),
            pltpu.VMEM_SHARED((256,), jnp.int32),
            pltpu.VMEM_SHARED((2 * CAND + 16,), jnp.float32),
            pltpu.VMEM_SHARED((2 * CAND + 16,), jnp.int32),
            pltpu.VMEM_SHARED((416,), jnp.int32),
            pltpu.SemaphoreType.DMA,
        ],
    )
    def sc_kernel(imp_hbm, thr_hbm, x_hbm, out_hbm,
                  imp_v, thr_v, cval_v, cidx_v,
                  cpos0, cpos1, cpos2, cpos3,
                  cnt_v, counts_l, call_v, myidx_v, gi_v, fpos_v,
                  rowid_v, rows_v,
                  counts_sh, cval_sh, cidx_sh, final_sh, sem):
        c = lax.axis_index("c")
        s = lax.axis_index("s")
        b_local = s // 8
        part = s % 8
        b = c * 2 + b_local
        iota = lax.iota(jnp.int32, 16)
        one = jnp.int32(1)
        zero = jnp.int32(0)

        pltpu.sync_copy(imp_hbm.at[pl.ds(b * L + part * 512, 512)], imp_v)
        pltpu.sync_copy(thr_hbm.at[b, pl.ds(0, 16)], thr_v)
        thr_bc = thr_v[...]

        def cbody(j, off):
            u = imp_v[pl.ds(j * 16, 16)]
            mi = jnp.where(u > thr_bc, one, zero)
            base = part * 512 + j * 16
            rel = zero
            pv = jnp.zeros((16,), jnp.float32)
            pg = jnp.zeros((16,), jnp.int32)
            for t in range(16):
                sel = mi[t]
                relv = jnp.full((16,), rel, jnp.int32)
                selv = jnp.full((16,), sel, jnp.int32)
                hiti = jnp.where(iota == relv, selv, jnp.zeros((16,), jnp.int32))
                pv = jnp.where(hiti > 0, jnp.full((16,), u[t], jnp.float32), pv)
                pg = jnp.where(hiti > 0, jnp.full((16,), base + t, jnp.int32), pg)
                rel = rel + sel
            cval_v[pl.ds(off, 16)] = pv
            cidx_v[pl.ds(off, 16)] = pg
            return off + rel

        mycnt = lax.fori_loop(0, 32, cbody, zero)

        cnt_v[...] = jnp.full((16,), mycnt, jnp.int32)
        pltpu.sync_copy(cnt_v, counts_sh.at[pl.ds(s * 16, 16)])
        plsc.subcore_barrier()
        pltpu.sync_copy(counts_sh, counts_l)
        my_off = zero
        c_total = zero
        for t in range(8):
            g_t = counts_l[pl.ds((b_local * 8 + t) * 16, 16)][0]
            my_off = my_off + jnp.where(t < part, g_t, zero)
            c_total = c_total + g_t
        c_total = jnp.minimum(c_total, jnp.int32(CAND))

        cpos = (cpos0, cpos1, cpos2, cpos3)
        for k in range(4):
            for t2 in range(8):
                lpos = jnp.full((16,), k * 128 + t2 * 16, jnp.int32) + iota
                tgt = jnp.full((16,), b_local * CAND + my_off,
                               jnp.int32) + lpos
                zv16 = jnp.zeros((16,), jnp.int32)
                ov16 = jnp.full((16,), 1, jnp.int32)
                oki = jnp.where(lpos < jnp.full((16,), mycnt, jnp.int32),
                                ov16, zv16)
                oki = oki * jnp.where(
                    lpos < jnp.full((16,), CAND - my_off, jnp.int32),
                    ov16, zv16)
                cpos[k][pl.ds(t2 * 16, 16)] = jnp.where(
                    oki > 0, tgt, jnp.full((16,), 2 * CAND + s, jnp.int32))
        for k in range(4):
            pltpu.sync_copy(cval_v.at[pl.ds(k * 128, 128)],
                            cval_sh.at[cpos[k]])
            pltpu.sync_copy(cidx_v.at[pl.ds(k * 128, 128)],
                            cidx_sh.at[cpos[k]])
        plsc.subcore_barrier()

        pltpu.sync_copy(cval_sh.at[pl.ds(b_local * CAND, CAND)],
                        call_v.at[pl.ds(0, CAND)])
        pltpu.sync_copy(cidx_sh.at[pl.ds(b_local * CAND + part * 64, 64)],
                        myidx_v)
        myv = []
        mypos = []
        for i in range(4):
            pos = part * 64 + i * 16
            myv.append(call_v[pl.ds(pos, 16)])
            mypos.append(jnp.full((16,), pos, jnp.int32) + iota)

        def rbody(m, ranks):
            u = call_v[pl.ds(m, 16)]
            w = jnp.full((16,), u[0], jnp.float32)
            mvec = jnp.full((16,), m, jnp.int32)
            zv16 = jnp.zeros((16,), jnp.int32)
            ov16 = jnp.full((16,), 1, jnp.int32)
            res = []
            for i in range(4):
                lt = jnp.where(mvec < mypos[i], ov16, zv16)
                tie = jnp.where(w == myv[i], lt, zv16)
                add = jnp.where(w > myv[i], ov16, tie)
                res.append(ranks[i] + add)
            return tuple(res)

        zv = jnp.zeros((16,), jnp.int32)
        ranks = lax.fori_loop(0, c_total, rbody, (zv, zv, zv, zv))

        for i in range(4):
            rk = ranks[i]
            zv16 = jnp.zeros((16,), jnp.int32)
            ov16 = jnp.full((16,), 1, jnp.int32)
            vi = jnp.where(rk < jnp.full((16,), NUM_TOKENS, jnp.int32),
                           ov16, zv16)
            vi = vi * jnp.where(
                mypos[i] < jnp.full((16,), c_total, jnp.int32), ov16, zv16)
            fpos = jnp.full((16,), b_local * NUM_TOKENS, jnp.int32) + rk
            spos = jnp.where(vi > 0, fpos,
                             jnp.full((16,), 400 + s, jnp.int32))
            gi = (myidx_v[pl.ds(i * 16, 16)]
                  + jnp.full((16,), b * L, jnp.int32))
            fpos_v[pl.ds(i * 16, 16)] = spos
            gi_v[pl.ds(i * 16, 16)] = gi
        pltpu.sync_copy(gi_v, final_sh.at[fpos_v])
        plsc.subcore_barrier()

        for k in range(2):
            chunk = s + 16 * k

            @pl.when(chunk < 25)
            def _gather_chunk(chunk=chunk):
                pltpu.sync_copy(final_sh.at[pl.ds(chunk * CHUNK, CHUNK)],
                                rowid_v)
                pltpu.async_copy(x_hbm.at[rowid_v], rows_v, sem).wait()
                out_row = c * 2 * NUM_TOKENS + chunk * CHUNK
                pltpu.sync_copy(rows_v, out_hbm.at[pl.ds(out_row, CHUNK)])

    return sc_kernel(imp_flat, thr, x2d)


def _matmul_body(sel_ref, w_ref, b_ref, out_ref):
    a = sel_ref[...]
    w = w_ref[...].astype(jnp.bfloat16)
    acc = lax.dot_general(
        a, w, (((1,), (1,)), ((), ())), preferred_element_type=jnp.float32)
    out_ref[...] = acc + b_ref[...]


def _project(sel2d, W, b):
    M, K = sel2d.shape
    N = W.shape[0]
    sel2d = sel2d.astype(jnp.bfloat16)
    BN = 1024
    return pl.pallas_call(
        _matmul_body,
        grid=(N // BN,),
        in_specs=[
            pl.BlockSpec((M, K), lambda j: (0, 0)),
            pl.BlockSpec((BN, K), lambda j: (j, 0)),
            pl.BlockSpec((1, BN), lambda j: (0, j)),
        ],
        out_specs=pl.BlockSpec((M, BN), lambda j: (0, j)),
        out_shape=jax.ShapeDtypeStruct((M, N), jnp.float32),
    )(sel2d, W, b[None, :])


def kernel(x, W, b):
    B, L, D = x.shape
    imp, thr = _importance(x)
    sel = _sc_select_gather(imp.reshape(B * L), thr, x.reshape(B * L, D))
    out = _project(sel, W, b)
    return out.reshape(B, NUM_TOKENS, W.shape[0])


# speedup vs baseline: 1.0838x; 1.0191x over previous
import functools

import jax
import jax.numpy as jnp
from jax import lax
from jax.experimental import pallas as pl
from jax.experimental.pallas import tpu as pltpu
from jax.experimental.pallas import tpu_sc as plsc

NUM_TOKENS = 200
NBINS = 128
CAND = 512


def _imp_body(x_ref, imp_ref, thr_ref):
    xb = x_ref[0]
    s = jnp.sum(xb, axis=0)
    imp = jnp.dot(xb, s[:, None],
                  preferred_element_type=jnp.float32)
    impv = imp[:, 0]
    imp_ref[...] = impv[None, None, :]
    m1 = jnp.min(imp)
    m2 = jnp.max(imp)
    span0 = m2 - m1
    lo = m1 - 0.001 * span0 - 1e-6
    span = m2 - lo
    t = lax.broadcasted_iota(jnp.int32, (1, NBINS), 1).astype(jnp.float32)
    edges = lo + span * t * (1.0 / NBINS)
    impb = jnp.broadcast_to(imp, (imp.shape[0], NBINS))
    edb = jnp.broadcast_to(edges, (imp.shape[0], NBINS))
    cmp = jnp.where(impb > edb, 1.0, 0.0)
    e = jnp.sum(cmp, axis=0, keepdims=True)
    tstar = jnp.sum(jnp.where(e >= NUM_TOKENS, 1.0, 0.0)) - 1.0
    tc = lo + span * tstar * (1.0 / NBINS)
    thr_ref[...] = jnp.full((1, 1, 128), tc, jnp.float32)


def _importance(x):
    B, L, D = x.shape
    imp, thr = pl.pallas_call(
        _imp_body,
        grid=(B,),
        in_specs=[pl.BlockSpec((1, L, D), lambda b: (b, 0, 0))],
        out_specs=[
            pl.BlockSpec((1, 1, L), lambda b: (b, 0, 0)),
            pl.BlockSpec((1, 1, 128), lambda b: (b, 0, 0)),
        ],
        out_shape=[
            jax.ShapeDtypeStruct((B, 1, L), jnp.float32),
            jax.ShapeDtypeStruct((B, 1, 128), jnp.float32),
        ],
    )(x)
    return imp.reshape(B, L), thr.reshape(B, 128)


def _sc_select_gather(imp_flat, thr, x2d):
    BL, D = x2d.shape
    B = thr.shape[0]
    L = BL // B
    CHUNK = 16

    mesh = plsc.VectorSubcoreMesh(core_axis_name="c", subcore_axis_name="s")

    @functools.partial(
        pl.kernel,
        mesh=mesh,
        out_type=jax.ShapeDtypeStruct((4 * NUM_TOKENS, D), jnp.float32),
        scratch_types=[
            pltpu.VMEM((512,), jnp.float32),
            pltpu.VMEM((16,), jnp.float32),
            pltpu.VMEM((528,), jnp.float32),
            pltpu.VMEM((528,), jnp.int32),
            pltpu.VMEM((128,), jnp.int32),
            pltpu.VMEM((128,), jnp.int32),
            pltpu.VMEM((128,), jnp.int32),
            pltpu.VMEM((128,), jnp.int32),
            pltpu.VMEM((16,), jnp.int32),
            pltpu.VMEM((256,), jnp.int32),
            pltpu.VMEM((528,), jnp.float32),
            pltpu.VMEM((64,), jnp.int32),
            pltpu.VMEM((64,), jnp.int32),
            pltpu.VMEM((64,), jnp.int32),
            pltpu.VMEM((CHUNK,), jnp.int32),
            pltpu.VMEM((CHUNK, D), jnp.float32),
            pltpu.VMEM_SHARED((256,), jnp.int32),
            pltpu.VMEM_SHARED((2 * CAND + 16,), jnp.float32),
            pltpu.VMEM_SHARED((2 * CAND + 16,), jnp.int32),
            pltpu.VMEM_SHARED((416,), jnp.int32),
            pltpu.SemaphoreType.DMA,
        ],
    )
    def sc_kernel(imp_hbm, thr_hbm, x_hbm, out_hbm,
                  imp_v, thr_v, cval_v, cidx_v,
                  cpos0, cpos1, cpos2, cpos3,
                  cnt_v, counts_l, call_v, myidx_v, gi_v, fpos_v,
                  rowid_v, rows_v,
                  counts_sh, cval_sh, cidx_sh, final_sh, sem):
        c = lax.axis_index("c")
        s = lax.axis_index("s")
        b_local = s // 8
        part = s % 8
        b = c * 2 + b_local
        iota = lax.iota(jnp.int32, 16)
        one = jnp.int32(1)
        zero = jnp.int32(0)

        pltpu.sync_copy(imp_hbm.at[pl.ds(b * L + part * 512, 512)], imp_v)
        pltpu.sync_copy(thr_hbm.at[b, pl.ds(0, 16)], thr_v)
        thr_bc = thr_v[...]

        def cbody(j, off):
            u = imp_v[pl.ds(j * 16, 16)]
            mi = jnp.where(u > thr_bc, one, zero)
            base = part * 512 + j * 16
            rel = zero
            pv = jnp.zeros((16,), jnp.float32)
            pg = jnp.zeros((16,), jnp.int32)
            for t in range(16):
                sel = mi[t]
                relv = jnp.full((16,), rel, jnp.int32)
                selv = jnp.full((16,), sel, jnp.int32)
                hiti = jnp.where(iota == relv, selv, jnp.zeros((16,), jnp.int32))
                pv = jnp.where(hiti > 0, jnp.full((16,), u[t], jnp.float32), pv)
                pg = jnp.where(hiti > 0, jnp.full((16,), base + t, jnp.int32), pg)
                rel = rel + sel
            cval_v[pl.ds(off, 16)] = pv
            cidx_v[pl.ds(off, 16)] = pg
            return off + rel

        mycnt = lax.fori_loop(0, 32, cbody, zero)

        cnt_v[...] = jnp.full((16,), mycnt, jnp.int32)
        pltpu.sync_copy(cnt_v, counts_sh.at[pl.ds(s * 16, 16)])
        plsc.subcore_barrier()
        pltpu.sync_copy(counts_sh, counts_l)
        my_off = zero
        c_total = zero
        for t in range(8):
            g_t = counts_l[pl.ds((b_local * 8 + t) * 16, 16)][0]
            my_off = my_off + jnp.where(t < part, g_t, zero)
            c_total = c_total + g_t
        c_total = jnp.minimum(c_total, jnp.int32(CAND))

        cpos = (cpos0, cpos1, cpos2, cpos3)
        for k in range(4):
            for t2 in range(8):
                lpos = jnp.full((16,), k * 128 + t2 * 16, jnp.int32) + iota
                tgt = jnp.full((16,), b_local * CAND + my_off,
                               jnp.int32) + lpos
                zv16 = jnp.zeros((16,), jnp.int32)
                ov16 = jnp.full((16,), 1, jnp.int32)
                oki = jnp.where(lpos < jnp.full((16,), mycnt, jnp.int32),
                                ov16, zv16)
                oki = oki * jnp.where(
                    lpos < jnp.full((16,), CAND - my_off, jnp.int32),
                    ov16, zv16)
                cpos[k][pl.ds(t2 * 16, 16)] = jnp.where(
                    oki > 0, tgt, jnp.full((16,), 2 * CAND + s, jnp.int32))
        for k in range(4):
            pltpu.sync_copy(cval_v.at[pl.ds(k * 128, 128)],
                            cval_sh.at[cpos[k]])
            pltpu.sync_copy(cidx_v.at[pl.ds(k * 128, 128)],
                            cidx_sh.at[cpos[k]])
        plsc.subcore_barrier()

        pltpu.sync_copy(cval_sh.at[pl.ds(b_local * CAND, CAND)],
                        call_v.at[pl.ds(0, CAND)])
        pltpu.sync_copy(cidx_sh.at[pl.ds(b_local * CAND + part * 64, 64)],
                        myidx_v)
        myv = []
        mypos = []
        for i in range(4):
            pos = part * 64 + i * 16
            myv.append(call_v[pl.ds(pos, 16)])
            mypos.append(jnp.full((16,), pos, jnp.int32) + iota)

        def rbody(m, ranks):
            u = call_v[pl.ds(m, 16)]
            w = jnp.full((16,), u[0], jnp.float32)
            mvec = jnp.full((16,), m, jnp.int32)
            zv16 = jnp.zeros((16,), jnp.int32)
            ov16 = jnp.full((16,), 1, jnp.int32)
            res = []
            for i in range(4):
                lt = jnp.where(mvec < mypos[i], ov16, zv16)
                tie = jnp.where(w == myv[i], lt, zv16)
                add = jnp.where(w > myv[i], ov16, tie)
                res.append(ranks[i] + add)
            return tuple(res)

        zv = jnp.zeros((16,), jnp.int32)
        ranks = lax.fori_loop(0, c_total, rbody, (zv, zv, zv, zv))

        for i in range(4):
            rk = ranks[i]
            zv16 = jnp.zeros((16,), jnp.int32)
            ov16 = jnp.full((16,), 1, jnp.int32)
            vi = jnp.where(rk < jnp.full((16,), NUM_TOKENS, jnp.int32),
                           ov16, zv16)
            vi = vi * jnp.where(
                mypos[i] < jnp.full((16,), c_total, jnp.int32), ov16, zv16)
            fpos = jnp.full((16,), b_local * NUM_TOKENS, jnp.int32) + rk
            spos = jnp.where(vi > 0, fpos,
                             jnp.full((16,), 400 + s, jnp.int32))
            gi = (myidx_v[pl.ds(i * 16, 16)]
                  + jnp.full((16,), b * L, jnp.int32))
            fpos_v[pl.ds(i * 16, 16)] = spos
            gi_v[pl.ds(i * 16, 16)] = gi
        pltpu.sync_copy(gi_v, final_sh.at[fpos_v])
        plsc.subcore_barrier()

        for k in range(2):
            chunk = s + 16 * k

            @pl.when(chunk < 25)
            def _gather_chunk(chunk=chunk):
                pltpu.sync_copy(final_sh.at[pl.ds(chunk * CHUNK, CHUNK)],
                                rowid_v)
                pltpu.async_copy(x_hbm.at[rowid_v], rows_v, sem).wait()
                out_row = c * 2 * NUM_TOKENS + chunk * CHUNK
                pltpu.sync_copy(rows_v, out_hbm.at[pl.ds(out_row, CHUNK)])

    return sc_kernel(imp_flat, thr, x2d)


def _matmul_body(sel_ref, w_ref, b_ref, out_ref):
    a = sel_ref[...].astype(jnp.bfloat16)
    w = w_ref[...].astype(jnp.bfloat16)
    acc = lax.dot_general(
        a, w, (((1,), (1,)), ((), ())), preferred_element_type=jnp.float32)
    out_ref[...] = acc + b_ref[...]


def _project(sel2d, W, b):
    M, K = sel2d.shape
    N = W.shape[0]
    BN = 512
    return pl.pallas_call(
        _matmul_body,
        grid=(N // BN,),
        in_specs=[
            pl.BlockSpec((M, K), lambda j: (0, 0)),
            pl.BlockSpec((BN, K), lambda j: (j, 0)),
            pl.BlockSpec((1, BN), lambda j: (0, j)),
        ],
        out_specs=pl.BlockSpec((M, BN), lambda j: (0, j)),
        out_shape=jax.ShapeDtypeStruct((M, N), jnp.float32),
    )(sel2d, W, b[None, :])


def kernel(x, W, b):
    B, L, D = x.shape
    imp, thr = _importance(x)
    sel = _sc_select_gather(imp.reshape(B * L), thr, x.reshape(B * L, D))
    out = _project(sel, W, b)
    return out.reshape(B, NUM_TOKENS, W.shape[0])
